# baseline (speedup 1.0000x reference)
import functools

import jax
import jax.numpy as jnp
from jax import lax
from jax.experimental import pallas as pl
from jax.experimental.pallas import tpu as pltpu
from jax.experimental.pallas import tpu_sc as plsc

N = 10000
NP = 10240
H = 256
HQ = 64
M = 160000
CS = 128
CH = 79
MP = 16 * CH * CS
RPT = NP // 16


def _u_body(h_ref, w_ref, b_ref, o_ref):
    p = jnp.dot(h_ref[...], w_ref[0], preferred_element_type=jnp.float32, precision=lax.Precision.HIGHEST)
    o_ref[...] = p + b_ref[0]


def _e_body(a_ref, w_ref, o_ref):
    o_ref[...] = jnp.dot(a_ref[...], w_ref[0], preferred_element_type=jnp.float32, precision=lax.Precision.HIGHEST)


def _ep_body(r_ref, d_ref, h_ref, w2_ref, b2_ref, w3a_ref, w3b_ref, b3_ref,
             w4_ref, b4_ref, g_ref, be_ref, o_ref):
    rb = jnp.concatenate([r_ref[0], r_ref[1], r_ref[2], r_ref[3]], axis=1)
    deg = d_ref[0][:, 0:1]
    agg = jnp.dot(rb, w2_ref[...], preferred_element_type=jnp.float32, precision=lax.Precision.HIGHEST)
    agg = agg + deg * b2_ref[...]
    z = jnp.dot(h_ref[...], w3a_ref[...], preferred_element_type=jnp.float32, precision=lax.Precision.HIGHEST)
    z = z + jnp.dot(agg, w3b_ref[...], preferred_element_type=jnp.float32, precision=lax.Precision.HIGHEST)
    z = z + b3_ref[...]
    a1 = jnp.maximum(z, 0.0)
    dh = jnp.dot(a1, w4_ref[...], preferred_element_type=jnp.float32, precision=lax.Precision.HIGHEST)
    dh = dh + b4_ref[...]
    y = h_ref[...] + dh
    mu = jnp.mean(y, axis=1, keepdims=True)
    d0 = y - mu
    var = jnp.mean(d0 * d0, axis=1, keepdims=True)
    o_ref[...] = d0 * lax.rsqrt(var + 1e-5) * g_ref[...] + be_ref[...]


def _sc_body(u_hbm, e_hbm, src_hbm, dst_hbm, r_hbm, d_hbm,
             sidx, didx, ubuf, ebuf, obuf, zbuf, acc, dacc, sem):
    c = lax.axis_index("c")
    s = lax.axis_index("s")

    def _init_row(i, _):
        obuf[i, :] = jnp.ones((16,), jnp.float32)
        zbuf[i, :] = jnp.zeros((16,), jnp.float32)
        return 0
    lax.fori_loop(0, CS, _init_row, 0)

    for phase in range(2):
        q = c * 2 + phase

        @plsc.parallel_loop(0, CS, unroll=4)
        def _zero_row(i):
            for j in range(HQ // 16):
                ubuf[i, pl.ds(j * 16, 16)] = jnp.zeros((16,), jnp.float32)
        for k in range(RPT // CS):
            pltpu.sync_copy(ubuf, acc.at[pl.ds(s * RPT + k * CS, CS), :])
            if phase == 0:
                pltpu.sync_copy(zbuf,
                                dacc.at[pl.ds(s * RPT + k * CS, CS), :])
        plsc.subcore_barrier()

        def _chunk(t, _):
            base = (s * CH + t) * CS
            pltpu.sync_copy(src_hbm.at[pl.ds(base, CS)], sidx)
            pltpu.sync_copy(dst_hbm.at[pl.ds(base, CS)], didx)
            off = q * NP
            for j in range(CS // 16):
                sidx[pl.ds(j * 16, 16)] = sidx[pl.ds(j * 16, 16)] + off

            pltpu.async_copy(u_hbm.at[sidx], ubuf, sem).wait()
            pltpu.sync_copy(e_hbm.at[pl.ds(q * MP + base, CS), :], ebuf)

            @plsc.parallel_loop(0, CS, unroll=4)
            def _row(i):
                for j in range(HQ // 16):
                    x = ubuf[i, pl.ds(j * 16, 16)] + ebuf[i, pl.ds(j * 16, 16)]
                    ubuf[i, pl.ds(j * 16, 16)] = jnp.maximum(x, 0.0)

            pltpu.sync_copy(ubuf, acc.at[didx], add=True)
            if phase == 0:
                pltpu.sync_copy(obuf, dacc.at[didx], add=True)
            return 0
        lax.fori_loop(0, CH, _chunk, 0)
        plsc.subcore_barrier()

        pltpu.sync_copy(acc.at[pl.ds(s * RPT, RPT), :],
                        r_hbm.at[q, pl.ds(s * RPT, RPT), :])
        if phase == 0:
            pltpu.sync_copy(dacc.at[pl.ds(s * RPT, RPT), :],
                            d_hbm.at[c, pl.ds(s * RPT, RPT), :])


def kernel(h, edge_index, edge_attr, W1, b1, W2, b2, W3, b3, W4, b4,
           gamma, beta):
    src = edge_index[0].astype(jnp.int32)
    dst = edge_index[1].astype(jnp.int32)
    pad = MP - M
    srcp = jnp.concatenate([src, jnp.full((pad,), N, jnp.int32)])
    dstp = jnp.concatenate([dst, jnp.full((pad,), N, jnp.int32)])
    h_pad = jnp.concatenate([h, jnp.zeros((NP - N, H), jnp.float32)], axis=0)
    ea_pad = jnp.concatenate(
        [edge_attr, jnp.zeros((pad, edge_attr.shape[1]), jnp.float32)], axis=0)

    w1a_q = W1[:H].reshape(H, 4, HQ).transpose(1, 0, 2)
    w1b_q = W1[H:].reshape(16, 4, HQ).transpose(1, 0, 2)
    b1_q = b1.reshape(4, 1, HQ)

    u_flat = pl.pallas_call(
        _u_body,
        grid=(NP // 128, 4),
        in_specs=[pl.BlockSpec((128, H), lambda i, q: (i, 0)),
                  pl.BlockSpec((1, H, HQ), lambda i, q: (q, 0, 0)),
                  pl.BlockSpec((1, 1, HQ), lambda i, q: (q, 0, 0))],
        out_specs=pl.BlockSpec((128, HQ), lambda i, q: (q * (NP // 128) + i, 0)),
        out_shape=jax.ShapeDtypeStruct((4 * NP, HQ), jnp.float32),
    )(h_pad, w1a_q, b1_q)

    EB = 512
    e_flat = pl.pallas_call(
        _e_body,
        grid=(MP // EB, 4),
        in_specs=[pl.BlockSpec((EB, 16), lambda i, q: (i, 0)),
                  pl.BlockSpec((1, 16, HQ), lambda i, q: (q, 0, 0))],
        out_specs=pl.BlockSpec((EB, HQ), lambda i, q: (q * (MP // EB) + i, 0)),
        out_shape=jax.ShapeDtypeStruct((4 * MP, HQ), jnp.float32),
    )(ea_pad, w1b_q)

    mesh = plsc.VectorSubcoreMesh(core_axis_name="c", subcore_axis_name="s")
    r, d = pl.kernel(
        _sc_body,
        mesh=mesh,
        compiler_params=pltpu.CompilerParams(use_tc_tiling_on_sc=False),
        out_type=[jax.ShapeDtypeStruct((4, NP, HQ), jnp.float32),
                  jax.ShapeDtypeStruct((2, NP, 16), jnp.float32)],
        scratch_types=[
            pltpu.VMEM((CS,), jnp.int32),
            pltpu.VMEM((CS,), jnp.int32),
            pltpu.VMEM((CS, HQ), jnp.float32),
            pltpu.VMEM((CS, HQ), jnp.float32),
            pltpu.VMEM((CS, 16), jnp.float32),
            pltpu.VMEM((CS, 16), jnp.float32),
            pltpu.VMEM_SHARED((NP, HQ), jnp.float32),
            pltpu.VMEM_SHARED((NP, 16), jnp.float32),
            pltpu.SemaphoreType.DMA,
        ],
    )(u_flat, e_flat, srcp, dstp)

    out = pl.pallas_call(
        _ep_body,
        grid=(NP // 128,),
        in_specs=[
            pl.BlockSpec((4, 128, HQ), lambda i: (0, i, 0)),
            pl.BlockSpec((2, 128, 16), lambda i: (0, i, 0)),
            pl.BlockSpec((128, H), lambda i: (i, 0)),
            pl.BlockSpec((H, H), lambda i: (0, 0)),
            pl.BlockSpec((1, H), lambda i: (0, 0)),
            pl.BlockSpec((H, H), lambda i: (0, 0)),
            pl.BlockSpec((H, H), lambda i: (0, 0)),
            pl.BlockSpec((1, H), lambda i: (0, 0)),
            pl.BlockSpec((H, H), lambda i: (0, 0)),
            pl.BlockSpec((1, H), lambda i: (0, 0)),
            pl.BlockSpec((1, H), lambda i: (0, 0)),
            pl.BlockSpec((1, H), lambda i: (0, 0)),
        ],
        out_specs=pl.BlockSpec((128, H), lambda i: (i, 0)),
        out_shape=jax.ShapeDtypeStruct((NP, H), jnp.float32),
    )(r, d, h_pad, W2, b2.reshape(1, H), W3[:H], W3[H:], b3.reshape(1, H),
      W4, b4.reshape(1, H), gamma.reshape(1, H), beta.reshape(1, H))
    return out[:N]


# speedup vs baseline: 1.1613x; 1.1613x over previous
import functools

import jax
import jax.numpy as jnp
from jax import lax
from jax.experimental import pallas as pl
from jax.experimental.pallas import tpu as pltpu
from jax.experimental.pallas import tpu_sc as plsc

N = 10000
NP = 10240
H = 256
HQ = 64
M = 160000
CS = 128
CH = 79
MP = 16 * CH * CS
RPT = NP // 16


def _u_body(h_ref, w_ref, b_ref, o_ref):
    p = jnp.dot(h_ref[...], w_ref[...], preferred_element_type=jnp.float32, precision=lax.Precision.HIGHEST)
    p = p + b_ref[...]
    o_ref[...] = jnp.stack([p[:, 0:64], p[:, 64:128],
                            p[:, 128:192], p[:, 192:256]], axis=0)


def _e_body(ev_ref, od_ref, w_ref, o_ref):
    pe = jnp.dot(ev_ref[...], w_ref[0], preferred_element_type=jnp.float32, precision=lax.Precision.HIGHEST)
    po = jnp.dot(od_ref[...], w_ref[0], preferred_element_type=jnp.float32, precision=lax.Precision.HIGHEST)
    o_ref[...] = jnp.concatenate([pe, po], axis=1)


def _ep_body(r_ref, d_ref, h_ref, w2_ref, b2_ref, w3a_ref, w3b_ref, b3_ref,
             w4_ref, b4_ref, g_ref, be_ref, o_ref):
    rb = jnp.concatenate([r_ref[0], r_ref[1], r_ref[2], r_ref[3]], axis=1)
    deg = d_ref[0][:, 0:1]
    agg = jnp.dot(rb, w2_ref[...], preferred_element_type=jnp.float32, precision=lax.Precision.HIGHEST)
    agg = agg + deg * b2_ref[...]
    z = jnp.dot(h_ref[...], w3a_ref[...], preferred_element_type=jnp.float32, precision=lax.Precision.HIGHEST)
    z = z + jnp.dot(agg, w3b_ref[...], preferred_element_type=jnp.float32, precision=lax.Precision.HIGHEST)
    z = z + b3_ref[...]
    a1 = jnp.maximum(z, 0.0)
    dh = jnp.dot(a1, w4_ref[...], preferred_element_type=jnp.float32, precision=lax.Precision.HIGHEST)
    dh = dh + b4_ref[...]
    y = h_ref[...] + dh
    mu = jnp.mean(y, axis=1, keepdims=True)
    d0 = y - mu
    var = jnp.mean(d0 * d0, axis=1, keepdims=True)
    o_ref[...] = d0 * lax.rsqrt(var + 1e-5) * g_ref[...] + be_ref[...]


def _sc_body(u_hbm, e_hbm, src_hbm, dst_hbm, r_hbm, d_hbm,
             sidx, didx, ubuf, ebuf, obuf, zbuf, acc, dacc, sem):
    c = lax.axis_index("c")
    s = lax.axis_index("s")

    def _init_row(i, _):
        obuf[i, :] = jnp.ones((16,), jnp.float32)
        zbuf[i, :] = jnp.zeros((16,), jnp.float32)
        return 0
    lax.fori_loop(0, CS, _init_row, 0)

    for phase in range(2):
        q = c * 2 + phase

        @plsc.parallel_loop(0, CS, unroll=4)
        def _zero_row(i):
            for j in range(HQ // 16):
                ubuf[i, pl.ds(j * 16, 16)] = jnp.zeros((16,), jnp.float32)
        for k in range(RPT // CS):
            pltpu.sync_copy(ubuf, acc.at[pl.ds(s * RPT + k * CS, CS), :])
            if phase == 0:
                pltpu.sync_copy(zbuf,
                                dacc.at[pl.ds(s * RPT + k * CS, CS), :])
        plsc.subcore_barrier()

        def _chunk(t, _):
            base = (s * CH + t) * CS
            pltpu.sync_copy(src_hbm.at[pl.ds(base, CS)], sidx)
            pltpu.sync_copy(dst_hbm.at[pl.ds(base, CS)], didx)
            off = q * NP
            for j in range(CS // 16):
                sidx[pl.ds(j * 16, 16)] = sidx[pl.ds(j * 16, 16)] + off

            pltpu.async_copy(u_hbm.at[sidx], ubuf, sem).wait()
            ehalf = q * (MP // 2) + (s * CH + t) * (CS // 2)
            pltpu.sync_copy(e_hbm.at[pl.ds(ehalf, CS // 2), :], ebuf)

            @plsc.parallel_loop(0, CS // 2, unroll=2)
            def _row(i2):
                for hp in range(2):
                    for j in range(HQ // 16):
                        x = (ubuf[2 * i2 + hp, pl.ds(j * 16, 16)]
                             + ebuf[i2, pl.ds(hp * HQ + j * 16, 16)])
                        ubuf[2 * i2 + hp, pl.ds(j * 16, 16)] = (
                            jnp.maximum(x, 0.0))

            pltpu.sync_copy(ubuf, acc.at[didx], add=True)
            if phase == 0:
                pltpu.sync_copy(obuf, dacc.at[didx], add=True)
            return 0
        lax.fori_loop(0, CH, _chunk, 0)
        plsc.subcore_barrier()

        pltpu.sync_copy(acc.at[pl.ds(s * RPT, RPT), :],
                        r_hbm.at[q, pl.ds(s * RPT, RPT), :])
        if phase == 0:
            pltpu.sync_copy(dacc.at[pl.ds(s * RPT, RPT), :],
                            d_hbm.at[c, pl.ds(s * RPT, RPT), :])


def kernel(h, edge_index, edge_attr, W1, b1, W2, b2, W3, b3, W4, b4,
           gamma, beta):
    src = edge_index[0].astype(jnp.int32)
    dst = edge_index[1].astype(jnp.int32)
    pad = MP - M
    srcp = jnp.concatenate([src, jnp.full((pad,), N, jnp.int32)])
    dstp = jnp.concatenate([dst, jnp.full((pad,), N, jnp.int32)])
    h_pad = jnp.concatenate([h, jnp.zeros((NP - N, H), jnp.float32)], axis=0)
    ea_pad = jnp.concatenate(
        [edge_attr, jnp.zeros((pad, edge_attr.shape[1]), jnp.float32)], axis=0)

    u = pl.pallas_call(
        _u_body,
        grid=(NP // 128,),
        in_specs=[pl.BlockSpec((128, H), lambda i: (i, 0)),
                  pl.BlockSpec((H, H), lambda i: (0, 0)),
                  pl.BlockSpec((1, H), lambda i: (0, 0))],
        out_specs=pl.BlockSpec((4, 128, HQ), lambda i: (0, i, 0)),
        out_shape=jax.ShapeDtypeStruct((4, NP, HQ), jnp.float32),
    )(h_pad, W1[:H], b1.reshape(1, H))
    u_flat = u.reshape(4 * NP, HQ)

    w1b_q = W1[H:].reshape(16, 4, HQ).transpose(1, 0, 2)
    ea_ev = ea_pad[0::2]
    ea_od = ea_pad[1::2]
    EB = 512
    MH = MP // 2
    e_pair = pl.pallas_call(
        _e_body,
        grid=(MH // EB, 4),
        in_specs=[pl.BlockSpec((EB, 16), lambda i, q: (i, 0)),
                  pl.BlockSpec((EB, 16), lambda i, q: (i, 0)),
                  pl.BlockSpec((1, 16, HQ), lambda i, q: (q, 0, 0))],
        out_specs=pl.BlockSpec((EB, 128), lambda i, q: (q * (MH // EB) + i, 0)),
        out_shape=jax.ShapeDtypeStruct((2 * MP, 128), jnp.float32),
    )(ea_ev, ea_od, w1b_q)

    mesh = plsc.VectorSubcoreMesh(core_axis_name="c", subcore_axis_name="s")
    r, d = pl.kernel(
        _sc_body,
        mesh=mesh,
        compiler_params=pltpu.CompilerParams(use_tc_tiling_on_sc=False),
        out_type=[jax.ShapeDtypeStruct((4, NP, HQ), jnp.float32),
                  jax.ShapeDtypeStruct((2, NP, 16), jnp.float32)],
        scratch_types=[
            pltpu.VMEM((CS,), jnp.int32),
            pltpu.VMEM((CS,), jnp.int32),
            pltpu.VMEM((CS, HQ), jnp.float32),
            pltpu.VMEM((CS // 2, 2 * HQ), jnp.float32),
            pltpu.VMEM((CS, 16), jnp.float32),
            pltpu.VMEM((CS, 16), jnp.float32),
            pltpu.VMEM_SHARED((NP, HQ), jnp.float32),
            pltpu.VMEM_SHARED((NP, 16), jnp.float32),
            pltpu.SemaphoreType.DMA,
        ],
    )(u_flat, e_pair, srcp, dstp)

    out = pl.pallas_call(
        _ep_body,
        grid=(NP // 128,),
        in_specs=[
            pl.BlockSpec((4, 128, HQ), lambda i: (0, i, 0)),
            pl.BlockSpec((2, 128, 16), lambda i: (0, i, 0)),
            pl.BlockSpec((128, H), lambda i: (i, 0)),
            pl.BlockSpec((H, H), lambda i: (0, 0)),
            pl.BlockSpec((1, H), lambda i: (0, 0)),
            pl.BlockSpec((H, H), lambda i: (0, 0)),
            pl.BlockSpec((H, H), lambda i: (0, 0)),
            pl.BlockSpec((1, H), lambda i: (0, 0)),
            pl.BlockSpec((H, H), lambda i: (0, 0)),
            pl.BlockSpec((1, H), lambda i: (0, 0)),
            pl.BlockSpec((1, H), lambda i: (0, 0)),
            pl.BlockSpec((1, H), lambda i: (0, 0)),
        ],
        out_specs=pl.BlockSpec((128, H), lambda i: (i, 0)),
        out_shape=jax.ShapeDtypeStruct((NP, H), jnp.float32),
    )(r, d, h_pad, W2, b2.reshape(1, H), W3[:H], W3[H:], b3.reshape(1, H),
      W4, b4.reshape(1, H), gamma.reshape(1, H), beta.reshape(1, H))
    return out[:N]


# speedup vs baseline: 1.2165x; 1.0475x over previous
import functools

import jax
import jax.numpy as jnp
from jax import lax
from jax.experimental import pallas as pl
from jax.experimental.pallas import tpu as pltpu
from jax.experimental.pallas import tpu_sc as plsc

N = 10000
NP = 10240
H = 256
HQ = 64
M = 160000
CS = 128
CH = 79
MP = 16 * CH * CS
RPT = NP // 16


def _u_body(h_ref, w_ref, b_ref, o_ref):
    p = jnp.dot(h_ref[...], w_ref[...], preferred_element_type=jnp.float32, precision=lax.Precision.HIGHEST)
    p = p + b_ref[...]
    o_ref[...] = jnp.stack([p[:, 0:64], p[:, 64:128],
                            p[:, 128:192], p[:, 192:256]], axis=0)


def _e_body(a_ref, w_ref, o_ref):
    p = jnp.dot(a_ref[...], w_ref[0], preferred_element_type=jnp.float32, precision=lax.Precision.HIGHEST)
    eb = p.shape[0]
    parts = [jnp.concatenate([p[c * CS:c * CS + CS // 2],
                              p[c * CS + CS // 2:(c + 1) * CS]], axis=1)
             for c in range(eb // CS)]
    o_ref[...] = jnp.concatenate(parts, axis=0)


def _ep_body(r_ref, d_ref, h_ref, w2_ref, b2_ref, w3a_ref, w3b_ref, b3_ref,
             w4_ref, b4_ref, g_ref, be_ref, o_ref):
    rb = jnp.concatenate([r_ref[0], r_ref[1], r_ref[2], r_ref[3]], axis=1)
    deg = d_ref[0][:, 0:1]
    agg = jnp.dot(rb, w2_ref[...], preferred_element_type=jnp.float32, precision=lax.Precision.HIGHEST)
    agg = agg + deg * b2_ref[...]
    z = jnp.dot(h_ref[...], w3a_ref[...], preferred_element_type=jnp.float32, precision=lax.Precision.HIGHEST)
    z = z + jnp.dot(agg, w3b_ref[...], preferred_element_type=jnp.float32, precision=lax.Precision.HIGHEST)
    z = z + b3_ref[...]
    a1 = jnp.maximum(z, 0.0)
    dh = jnp.dot(a1, w4_ref[...], preferred_element_type=jnp.float32, precision=lax.Precision.HIGHEST)
    dh = dh + b4_ref[...]
    y = h_ref[...] + dh
    mu = jnp.mean(y, axis=1, keepdims=True)
    d0 = y - mu
    var = jnp.mean(d0 * d0, axis=1, keepdims=True)
    o_ref[...] = d0 * lax.rsqrt(var + 1e-5) * g_ref[...] + be_ref[...]


def _sc_body(u_hbm, e_hbm, src_hbm, dst_hbm, r_hbm, d_hbm,
             sidx, didx, ubuf, ebuf, obuf, zbuf, acc, dacc, sem):
    c = lax.axis_index("c")
    s = lax.axis_index("s")

    def _init_row(i, _):
        obuf[i, :] = jnp.ones((16,), jnp.float32)
        zbuf[i, :] = jnp.zeros((16,), jnp.float32)
        return 0
    lax.fori_loop(0, CS, _init_row, 0)

    for phase in range(2):
        q = c * 2 + phase

        @plsc.parallel_loop(0, CS, unroll=4)
        def _zero_row(i):
            for j in range(HQ // 16):
                ubuf[i, pl.ds(j * 16, 16)] = jnp.zeros((16,), jnp.float32)
        for k in range(RPT // CS):
            pltpu.sync_copy(ubuf, acc.at[pl.ds(s * RPT + k * CS, CS), :])
            if phase == 0:
                pltpu.sync_copy(zbuf,
                                dacc.at[pl.ds(s * RPT + k * CS, CS), :])
        plsc.subcore_barrier()

        def _chunk(t, _):
            base = (s * CH + t) * CS
            pltpu.sync_copy(src_hbm.at[pl.ds(base, CS)], sidx)
            pltpu.sync_copy(dst_hbm.at[pl.ds(base, CS)], didx)
            off = q * NP
            for j in range(CS // 16):
                sidx[pl.ds(j * 16, 16)] = sidx[pl.ds(j * 16, 16)] + off

            pltpu.async_copy(u_hbm.at[sidx], ubuf, sem).wait()
            ehalf = q * (MP // 2) + (s * CH + t) * (CS // 2)
            pltpu.sync_copy(e_hbm.at[pl.ds(ehalf, CS // 2), :], ebuf)

            @plsc.parallel_loop(0, CS // 2, unroll=2)
            def _row(i2):
                for hp in range(2):
                    for j in range(HQ // 16):
                        x = (ubuf[i2 + hp * (CS // 2), pl.ds(j * 16, 16)]
                             + ebuf[i2, pl.ds(hp * HQ + j * 16, 16)])
                        ubuf[i2 + hp * (CS // 2), pl.ds(j * 16, 16)] = (
                            jnp.maximum(x, 0.0))

            pltpu.sync_copy(ubuf, acc.at[didx], add=True)
            if phase == 0:
                pltpu.sync_copy(obuf, dacc.at[didx], add=True)
            return 0
        lax.fori_loop(0, CH, _chunk, 0)
        plsc.subcore_barrier()

        pltpu.sync_copy(acc.at[pl.ds(s * RPT, RPT), :],
                        r_hbm.at[q, pl.ds(s * RPT, RPT), :])
        if phase == 0:
            pltpu.sync_copy(dacc.at[pl.ds(s * RPT, RPT), :],
                            d_hbm.at[c, pl.ds(s * RPT, RPT), :])


def kernel(h, edge_index, edge_attr, W1, b1, W2, b2, W3, b3, W4, b4,
           gamma, beta):
    src = edge_index[0].astype(jnp.int32)
    dst = edge_index[1].astype(jnp.int32)
    pad = MP - M
    srcp = jnp.concatenate([src, jnp.full((pad,), N, jnp.int32)])
    dstp = jnp.concatenate([dst, jnp.full((pad,), N, jnp.int32)])
    h_pad = jnp.concatenate([h, jnp.zeros((NP - N, H), jnp.float32)], axis=0)
    ea_pad = jnp.concatenate(
        [edge_attr, jnp.zeros((pad, edge_attr.shape[1]), jnp.float32)], axis=0)

    u = pl.pallas_call(
        _u_body,
        grid=(NP // 128,),
        in_specs=[pl.BlockSpec((128, H), lambda i: (i, 0)),
                  pl.BlockSpec((H, H), lambda i: (0, 0)),
                  pl.BlockSpec((1, H), lambda i: (0, 0))],
        out_specs=pl.BlockSpec((4, 128, HQ), lambda i: (0, i, 0)),
        out_shape=jax.ShapeDtypeStruct((4, NP, HQ), jnp.float32),
    )(h_pad, W1[:H], b1.reshape(1, H))
    u_flat = u.reshape(4 * NP, HQ)

    w1b_q = W1[H:].reshape(16, 4, HQ).transpose(1, 0, 2)
    EB = 512
    MH = MP // 2
    e_pair = pl.pallas_call(
        _e_body,
        grid=(MP // EB, 4),
        in_specs=[pl.BlockSpec((EB, 16), lambda i, q: (i, 0)),
                  pl.BlockSpec((1, 16, HQ), lambda i, q: (q, 0, 0))],
        out_specs=pl.BlockSpec((EB // 2, 128),
                               lambda i, q: (q * (MP // EB) + i, 0)),
        out_shape=jax.ShapeDtypeStruct((2 * MP, 128), jnp.float32),
    )(ea_pad, w1b_q)

    mesh = plsc.VectorSubcoreMesh(core_axis_name="c", subcore_axis_name="s")
    r, d = pl.kernel(
        _sc_body,
        mesh=mesh,
        compiler_params=pltpu.CompilerParams(use_tc_tiling_on_sc=False),
        out_type=[jax.ShapeDtypeStruct((4, NP, HQ), jnp.float32),
                  jax.ShapeDtypeStruct((2, NP, 16), jnp.float32)],
        scratch_types=[
            pltpu.VMEM((CS,), jnp.int32),
            pltpu.VMEM((CS,), jnp.int32),
            pltpu.VMEM((CS, HQ), jnp.float32),
            pltpu.VMEM((CS // 2, 2 * HQ), jnp.float32),
            pltpu.VMEM((CS, 16), jnp.float32),
            pltpu.VMEM((CS, 16), jnp.float32),
            pltpu.VMEM_SHARED((NP, HQ), jnp.float32),
            pltpu.VMEM_SHARED((NP, 16), jnp.float32),
            pltpu.SemaphoreType.DMA,
        ],
    )(u_flat, e_pair, srcp, dstp)

    out = pl.pallas_call(
        _ep_body,
        grid=(NP // 128,),
        in_specs=[
            pl.BlockSpec((4, 128, HQ), lambda i: (0, i, 0)),
            pl.BlockSpec((2, 128, 16), lambda i: (0, i, 0)),
            pl.BlockSpec((128, H), lambda i: (i, 0)),
            pl.BlockSpec((H, H), lambda i: (0, 0)),
            pl.BlockSpec((1, H), lambda i: (0, 0)),
            pl.BlockSpec((H, H), lambda i: (0, 0)),
            pl.BlockSpec((H, H), lambda i: (0, 0)),
            pl.BlockSpec((1, H), lambda i: (0, 0)),
            pl.BlockSpec((H, H), lambda i: (0, 0)),
            pl.BlockSpec((1, H), lambda i: (0, 0)),
            pl.BlockSpec((1, H), lambda i: (0, 0)),
            pl.BlockSpec((1, H), lambda i: (0, 0)),
        ],
        out_specs=pl.BlockSpec((128, H), lambda i: (i, 0)),
        out_shape=jax.ShapeDtypeStruct((NP, H), jnp.float32),
    )(r, d, h_pad, W2, b2.reshape(1, H), W3[:H], W3[H:], b3.reshape(1, H),
      W4, b4.reshape(1, H), gamma.reshape(1, H), beta.reshape(1, H))
    return out[:N]


# speedup vs baseline: 1.2368x; 1.0166x over previous
import functools

import jax
import jax.numpy as jnp
from jax import lax
from jax.experimental import pallas as pl
from jax.experimental.pallas import tpu as pltpu
from jax.experimental.pallas import tpu_sc as plsc

N = 10000
NP = 10240
H = 256
HQ = 64
M = 160000
CS = 128
CH = 79
MP = 16 * CH * CS
RPT = NP // 16


def _u_body(h_ref, w_ref, b_ref, o_ref):
    p = jnp.dot(h_ref[...], w_ref[...], preferred_element_type=jnp.float32, precision=lax.Precision.HIGHEST)
    p = p + b_ref[...]
    o_ref[...] = jnp.stack([p[:, 0:64], p[:, 64:128],
                            p[:, 128:192], p[:, 192:256]], axis=0)


def _e_body(a_ref, w_ref, o_ref):
    p2 = jnp.dot(a_ref[...], w_ref[0], preferred_element_type=jnp.float32, precision=lax.Precision.HIGHEST)
    eb = p2.shape[0]
    lane = lax.broadcasted_iota(jnp.int32, (CS // 2, 128), 1)
    parts = [jnp.where(lane < HQ, p2[c * CS:c * CS + CS // 2],
                       p2[c * CS + CS // 2:(c + 1) * CS])
             for c in range(eb // CS)]
    o_ref[...] = jnp.concatenate(parts, axis=0)


def _ep_body(r_ref, d_ref, h_ref, w2_ref, b2_ref, w3a_ref, w3b_ref, b3_ref,
             w4_ref, b4_ref, g_ref, be_ref, o_ref):
    rb = jnp.concatenate([r_ref[0], r_ref[1], r_ref[2], r_ref[3]], axis=1)
    deg = d_ref[0][:, 0:1]
    agg = jnp.dot(rb, w2_ref[...], preferred_element_type=jnp.float32, precision=lax.Precision.HIGHEST)
    agg = agg + deg * b2_ref[...]
    z = jnp.dot(h_ref[...], w3a_ref[...], preferred_element_type=jnp.float32, precision=lax.Precision.HIGHEST)
    z = z + jnp.dot(agg, w3b_ref[...], preferred_element_type=jnp.float32, precision=lax.Precision.HIGHEST)
    z = z + b3_ref[...]
    a1 = jnp.maximum(z, 0.0)
    dh = jnp.dot(a1, w4_ref[...], preferred_element_type=jnp.float32, precision=lax.Precision.HIGHEST)
    dh = dh + b4_ref[...]
    y = h_ref[...] + dh
    mu = jnp.mean(y, axis=1, keepdims=True)
    d0 = y - mu
    var = jnp.mean(d0 * d0, axis=1, keepdims=True)
    o_ref[...] = d0 * lax.rsqrt(var + 1e-5) * g_ref[...] + be_ref[...]


def _sc_body(u_hbm, e_hbm, src_hbm, dst_hbm, r_hbm, d_hbm,
             sidx, didx, ubuf, ebuf, obuf, zbuf, acc, dacc, sem):
    c = lax.axis_index("c")
    s = lax.axis_index("s")

    def _init_row(i, _):
        obuf[i, :] = jnp.ones((16,), jnp.float32)
        zbuf[i, :] = jnp.zeros((16,), jnp.float32)
        return 0
    lax.fori_loop(0, CS, _init_row, 0)

    for phase in range(2):
        q = c * 2 + phase

        @plsc.parallel_loop(0, CS, unroll=4)
        def _zero_row(i):
            for j in range(HQ // 16):
                ubuf[i, pl.ds(j * 16, 16)] = jnp.zeros((16,), jnp.float32)
        for k in range(RPT // CS):
            pltpu.sync_copy(ubuf, acc.at[pl.ds(s * RPT + k * CS, CS), :])
            if phase == 0:
                pltpu.sync_copy(zbuf,
                                dacc.at[pl.ds(s * RPT + k * CS, CS), :])
        plsc.subcore_barrier()

        def _chunk(t, _):
            base = (s * CH + t) * CS
            pltpu.sync_copy(src_hbm.at[pl.ds(base, CS)], sidx)
            pltpu.sync_copy(dst_hbm.at[pl.ds(base, CS)], didx)
            off = q * NP
            for j in range(CS // 16):
                sidx[pl.ds(j * 16, 16)] = sidx[pl.ds(j * 16, 16)] + off

            pltpu.async_copy(u_hbm.at[sidx], ubuf, sem).wait()
            ehalf = q * (MP // 2) + (s * CH + t) * (CS // 2)
            pltpu.sync_copy(e_hbm.at[pl.ds(ehalf, CS // 2), :], ebuf)

            @plsc.parallel_loop(0, CS // 2, unroll=2)
            def _row(i2):
                for hp in range(2):
                    for j in range(HQ // 16):
                        x = (ubuf[i2 + hp * (CS // 2), pl.ds(j * 16, 16)]
                             + ebuf[i2, pl.ds(hp * HQ + j * 16, 16)])
                        ubuf[i2 + hp * (CS // 2), pl.ds(j * 16, 16)] = (
                            jnp.maximum(x, 0.0))

            pltpu.sync_copy(ubuf, acc.at[didx], add=True)
            if phase == 0:
                pltpu.sync_copy(obuf, dacc.at[didx], add=True)
            return 0
        lax.fori_loop(0, CH, _chunk, 0)
        plsc.subcore_barrier()

        pltpu.sync_copy(acc.at[pl.ds(s * RPT, RPT), :],
                        r_hbm.at[q, pl.ds(s * RPT, RPT), :])
        if phase == 0:
            pltpu.sync_copy(dacc.at[pl.ds(s * RPT, RPT), :],
                            d_hbm.at[c, pl.ds(s * RPT, RPT), :])


def kernel(h, edge_index, edge_attr, W1, b1, W2, b2, W3, b3, W4, b4,
           gamma, beta):
    src = edge_index[0].astype(jnp.int32)
    dst = edge_index[1].astype(jnp.int32)
    pad = MP - M
    srcp = jnp.concatenate([src, jnp.full((pad,), N, jnp.int32)])
    dstp = jnp.concatenate([dst, jnp.full((pad,), N, jnp.int32)])
    h_pad = jnp.concatenate([h, jnp.zeros((NP - N, H), jnp.float32)], axis=0)
    ea_pad = jnp.concatenate(
        [edge_attr, jnp.zeros((pad, edge_attr.shape[1]), jnp.float32)], axis=0)

    u = pl.pallas_call(
        _u_body,
        grid=(NP // 128,),
        in_specs=[pl.BlockSpec((128, H), lambda i: (i, 0)),
                  pl.BlockSpec((H, H), lambda i: (0, 0)),
                  pl.BlockSpec((1, H), lambda i: (0, 0))],
        out_specs=pl.BlockSpec((4, 128, HQ), lambda i: (0, i, 0)),
        out_shape=jax.ShapeDtypeStruct((4, NP, HQ), jnp.float32),
    )(h_pad, W1[:H], b1.reshape(1, H))
    u_flat = u.reshape(4 * NP, HQ)

    w1b_q = W1[H:].reshape(16, 4, HQ).transpose(1, 0, 2)
    w1b_d = jnp.concatenate([w1b_q, w1b_q], axis=2)
    EB = 512
    MH = MP // 2
    e_pair = pl.pallas_call(
        _e_body,
        grid=(MP // EB, 4),
        in_specs=[pl.BlockSpec((EB, 16), lambda i, q: (i, 0)),
                  pl.BlockSpec((1, 16, 128), lambda i, q: (q, 0, 0))],
        out_specs=pl.BlockSpec((EB // 2, 128),
                               lambda i, q: (q * (MP // EB) + i, 0)),
        out_shape=jax.ShapeDtypeStruct((2 * MP, 128), jnp.float32),
    )(ea_pad, w1b_d)

    mesh = plsc.VectorSubcoreMesh(core_axis_name="c", subcore_axis_name="s")
    r, d = pl.kernel(
        _sc_body,
        mesh=mesh,
        compiler_params=pltpu.CompilerParams(use_tc_tiling_on_sc=False),
        out_type=[jax.ShapeDtypeStruct((4, NP, HQ), jnp.float32),
                  jax.ShapeDtypeStruct((2, NP, 16), jnp.float32)],
        scratch_types=[
            pltpu.VMEM((CS,), jnp.int32),
            pltpu.VMEM((CS,), jnp.int32),
            pltpu.VMEM((CS, HQ), jnp.float32),
            pltpu.VMEM((CS // 2, 2 * HQ), jnp.float32),
            pltpu.VMEM((CS, 16), jnp.float32),
            pltpu.VMEM((CS, 16), jnp.float32),
            pltpu.VMEM_SHARED((NP, HQ), jnp.float32),
            pltpu.VMEM_SHARED((NP, 16), jnp.float32),
            pltpu.SemaphoreType.DMA,
        ],
    )(u_flat, e_pair, srcp, dstp)

    out = pl.pallas_call(
        _ep_body,
        grid=(NP // 128,),
        in_specs=[
            pl.BlockSpec((4, 128, HQ), lambda i: (0, i, 0)),
            pl.BlockSpec((2, 128, 16), lambda i: (0, i, 0)),
            pl.BlockSpec((128, H), lambda i: (i, 0)),
            pl.BlockSpec((H, H), lambda i: (0, 0)),
            pl.BlockSpec((1, H), lambda i: (0, 0)),
            pl.BlockSpec((H, H), lambda i: (0, 0)),
            pl.BlockSpec((H, H), lambda i: (0, 0)),
            pl.BlockSpec((1, H), lambda i: (0, 0)),
            pl.BlockSpec((H, H), lambda i: (0, 0)),
            pl.BlockSpec((1, H), lambda i: (0, 0)),
            pl.BlockSpec((1, H), lambda i: (0, 0)),
            pl.BlockSpec((1, H), lambda i: (0, 0)),
        ],
        out_specs=pl.BlockSpec((128, H), lambda i: (i, 0)),
        out_shape=jax.ShapeDtypeStruct((NP, H), jnp.float32),
    )(r, d, h_pad, W2, b2.reshape(1, H), W3[:H], W3[H:], b3.reshape(1, H),
      W4, b4.reshape(1, H), gamma.reshape(1, H), beta.reshape(1, H))
    return out[:N]


# speedup vs baseline: 1.5358x; 1.2417x over previous
import functools

import jax
import jax.numpy as jnp
from jax import lax
from jax.experimental import pallas as pl
from jax.experimental.pallas import tpu as pltpu
from jax.experimental.pallas import tpu_sc as plsc

N = 10000
NP = 10240
H = 256
HQ = 64
M = 160000
CS = 128
CH = 80
MP = 16 * CH * CS
K = 8
RPT = NP // 16


def _u_body(h_ref, w_ref, b_ref, o_ref):
    p = jnp.dot(h_ref[...], w_ref[...], preferred_element_type=jnp.float32, precision=lax.Precision.HIGHEST)
    p = p + b_ref[...]
    o_ref[...] = jnp.stack([p[:, 0:64], p[:, 64:128],
                            p[:, 128:192], p[:, 192:256]], axis=0)


def _e_body(a_ref, w_ref, o_ref):
    p2 = jnp.dot(a_ref[...], w_ref[0], preferred_element_type=jnp.float32)
    eb = p2.shape[0]
    lane = lax.broadcasted_iota(jnp.int32, (CS // 2, 128), 1)
    parts = [jnp.where(lane < HQ, p2[c * CS:c * CS + CS // 2],
                       p2[c * CS + CS // 2:(c + 1) * CS])
             for c in range(eb // CS)]
    o_ref[...] = jnp.concatenate(parts, axis=0)


def _ep_body(r_ref, d_ref, h_ref, w2_ref, b2_ref, w3a_ref, w3b_ref, b3_ref,
             w4_ref, b4_ref, g_ref, be_ref, o_ref):
    rb = jnp.concatenate([r_ref[0], r_ref[1], r_ref[2], r_ref[3]], axis=1)
    deg = d_ref[0][:, 0:1]
    agg = jnp.dot(rb, w2_ref[...], preferred_element_type=jnp.float32, precision=lax.Precision.HIGHEST)
    agg = agg + deg * b2_ref[...]
    z = jnp.dot(h_ref[...], w3a_ref[...], preferred_element_type=jnp.float32, precision=lax.Precision.HIGHEST)
    z = z + jnp.dot(agg, w3b_ref[...], preferred_element_type=jnp.float32, precision=lax.Precision.HIGHEST)
    z = z + b3_ref[...]
    a1 = jnp.maximum(z, 0.0)
    dh = jnp.dot(a1, w4_ref[...], preferred_element_type=jnp.float32, precision=lax.Precision.HIGHEST)
    dh = dh + b4_ref[...]
    y = h_ref[...] + dh
    mu = jnp.mean(y, axis=1, keepdims=True)
    d0 = y - mu
    var = jnp.mean(d0 * d0, axis=1, keepdims=True)
    o_ref[...] = d0 * lax.rsqrt(var + 1e-5) * g_ref[...] + be_ref[...]


def _sc_body(u_hbm, e_hbm, srcq_hbm, dst3_hbm, r_hbm, d_hbm,
             sidx, didx, ubuf0, ubuf1, ebuf0, ebuf1, obuf, zbuf,
             acc, dacc, sem0, sem1):
    c = lax.axis_index("c")
    s = lax.axis_index("s")

    @plsc.parallel_loop(0, CS, unroll=4)
    def _init_row(i):
        obuf[i, :] = jnp.ones((16,), jnp.float32)
        zbuf[i, :] = jnp.zeros((16,), jnp.float32)

    ubufs = (ubuf0, ubuf1)
    ebufs = (ebuf0, ebuf1)
    sems = (sem0, sem1)

    for phase in range(2):
        q = c * 2 + phase

        @plsc.parallel_loop(0, CS, unroll=4)
        def _zero_row(i):
            for j in range(HQ // 16):
                ubuf0[i, pl.ds(j * 16, 16)] = jnp.zeros((16,), jnp.float32)
        for k in range(RPT // CS):
            pltpu.sync_copy(ubuf0, acc.at[pl.ds(s * RPT + k * CS, CS), :])
            if phase == 0:
                pltpu.sync_copy(zbuf,
                                dacc.at[pl.ds(s * RPT + k * CS, CS), :])
        plsc.subcore_barrier()

        def _group(g, _):
            chunk0 = s * CH + g * K
            base = chunk0 * CS
            pltpu.sync_copy(srcq_hbm.at[pl.ds(q * MP + base, K * CS)], sidx)
            pltpu.sync_copy(dst3_hbm.at[pl.ds(chunk0, K), :, :], didx)

            def _issue(j):
                slot = j & 1
                gh = pltpu.async_copy(u_hbm.at[sidx.at[pl.ds(j * CS, CS)]],
                                      ubufs[slot], sems[slot])
                erow = q * (MP // 2) + (base + j * CS) // 2
                eh = pltpu.async_copy(e_hbm.at[pl.ds(erow, CS // 2), :],
                                      ebufs[slot], sems[slot])
                return gh, eh

            hs = [None, None]
            hs[0] = _issue(0)
            for j in range(K):
                slot = j & 1
                if j + 1 < K:
                    hs[(j + 1) & 1] = _issue(j + 1)
                gh, eh = hs[slot]
                gh.wait()
                eh.wait()
                ub = ubufs[slot]
                eb = ebufs[slot]

                @plsc.parallel_loop(0, CS // 2, unroll=2)
                def _row(i2, _ub=ub, _eb=eb):
                    for hp in range(2):
                        for jj in range(HQ // 16):
                            x = (_ub[i2 + hp * (CS // 2), pl.ds(jj * 16, 16)]
                                 + _eb[i2, pl.ds(hp * HQ + jj * 16, 16)])
                            _ub[i2 + hp * (CS // 2), pl.ds(jj * 16, 16)] = (
                                jnp.maximum(x, 0.0))

                pltpu.sync_copy(ub, acc.at[didx.at[j, 0]], add=True)
                if phase == 0:
                    pltpu.sync_copy(obuf, dacc.at[didx.at[j, 0]], add=True)
            return 0
        lax.fori_loop(0, CH // K, _group, 0)
        plsc.subcore_barrier()

        pltpu.sync_copy(acc.at[pl.ds(s * RPT, RPT), :],
                        r_hbm.at[q, pl.ds(s * RPT, RPT), :])
        if phase == 0:
            pltpu.sync_copy(dacc.at[pl.ds(s * RPT, RPT), :],
                            d_hbm.at[c, pl.ds(s * RPT, RPT), :])


def kernel(h, edge_index, edge_attr, W1, b1, W2, b2, W3, b3, W4, b4,
           gamma, beta):
    src = edge_index[0].astype(jnp.int32)
    dst = edge_index[1].astype(jnp.int32)
    pad = MP - M
    srcp = jnp.concatenate([src, jnp.full((pad,), N, jnp.int32)])
    dstp = jnp.concatenate([dst, jnp.full((pad,), N, jnp.int32)])
    h_pad = jnp.concatenate([h, jnp.zeros((NP - N, H), jnp.float32)], axis=0)
    ea_pad = jnp.concatenate(
        [edge_attr, jnp.zeros((pad, edge_attr.shape[1]), jnp.float32)], axis=0)

    u = pl.pallas_call(
        _u_body,
        grid=(NP // 128,),
        in_specs=[pl.BlockSpec((128, H), lambda i: (i, 0)),
                  pl.BlockSpec((H, H), lambda i: (0, 0)),
                  pl.BlockSpec((1, H), lambda i: (0, 0))],
        out_specs=pl.BlockSpec((4, 128, HQ), lambda i: (0, i, 0)),
        out_shape=jax.ShapeDtypeStruct((4, NP, HQ), jnp.float32),
    )(h_pad, W1[:H], b1.reshape(1, H))
    u_flat = u.reshape(4 * NP, HQ)

    w1b_q = W1[H:].reshape(16, 4, HQ).transpose(1, 0, 2)
    w1b_d = jnp.concatenate([w1b_q, w1b_q], axis=2)
    EB = 512
    MH = MP // 2
    e_pair = pl.pallas_call(
        _e_body,
        grid=(MP // EB, 4),
        in_specs=[pl.BlockSpec((EB, 16), lambda i, q: (i, 0)),
                  pl.BlockSpec((1, 16, 128), lambda i, q: (q, 0, 0))],
        out_specs=pl.BlockSpec((EB // 2, 128),
                               lambda i, q: (q * (MP // EB) + i, 0)),
        out_shape=jax.ShapeDtypeStruct((2 * MP, 128), jnp.float32),
    )(ea_pad, w1b_d)

    srcq = jnp.concatenate([srcp + qq * NP for qq in range(4)])
    dst3 = dstp.reshape(MP // CS, 1, CS)

    mesh = plsc.VectorSubcoreMesh(core_axis_name="c", subcore_axis_name="s")
    r, d = pl.kernel(
        _sc_body,
        mesh=mesh,
        compiler_params=pltpu.CompilerParams(use_tc_tiling_on_sc=False),
        out_type=[jax.ShapeDtypeStruct((4, NP, HQ), jnp.float32),
                  jax.ShapeDtypeStruct((2, NP, 16), jnp.float32)],
        scratch_types=[
            pltpu.VMEM((K * CS,), jnp.int32),
            pltpu.VMEM((K, 1, CS), jnp.int32),
            pltpu.VMEM((CS, HQ), jnp.float32),
            pltpu.VMEM((CS, HQ), jnp.float32),
            pltpu.VMEM((CS // 2, 2 * HQ), jnp.float32),
            pltpu.VMEM((CS // 2, 2 * HQ), jnp.float32),
            pltpu.VMEM((CS, 16), jnp.float32),
            pltpu.VMEM((CS, 16), jnp.float32),
            pltpu.VMEM_SHARED((NP, HQ), jnp.float32),
            pltpu.VMEM_SHARED((NP, 16), jnp.float32),
            pltpu.SemaphoreType.DMA,
            pltpu.SemaphoreType.DMA,
        ],
    )(u_flat, e_pair, srcq, dst3)

    out = pl.pallas_call(
        _ep_body,
        grid=(NP // 128,),
        in_specs=[
            pl.BlockSpec((4, 128, HQ), lambda i: (0, i, 0)),
            pl.BlockSpec((2, 128, 16), lambda i: (0, i, 0)),
            pl.BlockSpec((128, H), lambda i: (i, 0)),
            pl.BlockSpec((H, H), lambda i: (0, 0)),
            pl.BlockSpec((1, H), lambda i: (0, 0)),
            pl.BlockSpec((H, H), lambda i: (0, 0)),
            pl.BlockSpec((H, H), lambda i: (0, 0)),
            pl.BlockSpec((1, H), lambda i: (0, 0)),
            pl.BlockSpec((H, H), lambda i: (0, 0)),
            pl.BlockSpec((1, H), lambda i: (0, 0)),
            pl.BlockSpec((1, H), lambda i: (0, 0)),
            pl.BlockSpec((1, H), lambda i: (0, 0)),
        ],
        out_specs=pl.BlockSpec((128, H), lambda i: (i, 0)),
        out_shape=jax.ShapeDtypeStruct((NP, H), jnp.float32),
    )(r, d, h_pad, W2, b2.reshape(1, H), W3[:H], W3[H:], b3.reshape(1, H),
      W4, b4.reshape(1, H), gamma.reshape(1, H), beta.reshape(1, H))
    return out[:N]


# speedup vs baseline: 1.9743x; 1.2855x over previous
import functools

import jax
import jax.numpy as jnp
from jax import lax
from jax.experimental import pallas as pl
from jax.experimental.pallas import tpu as pltpu
from jax.experimental.pallas import tpu_sc as plsc

N = 10000
NP = 10240
H = 256
HQ = 64
M = 160000
CS = 128
CH = 80
MP = 16 * CH * CS
K = 8
RPT = NP // 16


def _u_body(h_ref, w_ref, b_ref, o_ref):
    p = jnp.dot(h_ref[...], w_ref[...], preferred_element_type=jnp.float32, precision=lax.Precision.HIGHEST)
    p = p + b_ref[...]
    o_ref[...] = jnp.stack([p[:, 0:64], p[:, 64:128],
                            p[:, 128:192], p[:, 192:256]], axis=0)


def _e_body(a_ref, w_ref, o_ref):
    o_ref[...] = jnp.dot(a_ref[...], w_ref[0],
                         preferred_element_type=jnp.float32)


def _ep_body(r_ref, d_ref, h_ref, w2_ref, b2_ref, w3a_ref, w3b_ref, b3_ref,
             w4_ref, b4_ref, g_ref, be_ref, o_ref):
    rb = jnp.concatenate([r_ref[0], r_ref[1], r_ref[2], r_ref[3]], axis=1)
    deg = d_ref[0][:, 0:1]
    agg = jnp.dot(rb, w2_ref[...], preferred_element_type=jnp.float32, precision=lax.Precision.HIGHEST)
    agg = agg + deg * b2_ref[...]
    z = jnp.dot(h_ref[...], w3a_ref[...], preferred_element_type=jnp.float32, precision=lax.Precision.HIGHEST)
    z = z + jnp.dot(agg, w3b_ref[...], preferred_element_type=jnp.float32, precision=lax.Precision.HIGHEST)
    z = z + b3_ref[...]
    a1 = jnp.maximum(z, 0.0)
    dh = jnp.dot(a1, w4_ref[...], preferred_element_type=jnp.float32, precision=lax.Precision.HIGHEST)
    dh = dh + b4_ref[...]
    y = h_ref[...] + dh
    mu = jnp.mean(y, axis=1, keepdims=True)
    d0 = y - mu
    var = jnp.mean(d0 * d0, axis=1, keepdims=True)
    o_ref[...] = d0 * lax.rsqrt(var + 1e-5) * g_ref[...] + be_ref[...]


def _sc_body(u_hbm, e_hbm, srcq_hbm, dst3_hbm, r_hbm, d_hbm,
             sidx, didx, ubuf0, ubuf1, ebuf0, ebuf1, obuf, zbuf,
             acc, dacc, sem0, sem1):
    c = lax.axis_index("c")
    s = lax.axis_index("s")

    @plsc.parallel_loop(0, CS, unroll=4)
    def _init_row(i):
        obuf[i, :] = jnp.ones((16,), jnp.float32)
        zbuf[i, :] = jnp.zeros((16,), jnp.float32)

    ubufs = (ubuf0, ubuf1)
    ebufs = (ebuf0, ebuf1)
    sems = (sem0, sem1)

    for phase in range(2):
        q = c * 2 + phase

        @plsc.parallel_loop(0, CS, unroll=4)
        def _zero_row(i):
            for j in range(HQ // 16):
                ubuf0[i, pl.ds(j * 16, 16)] = jnp.zeros((16,), jnp.float32)
        for k in range(RPT // CS):
            pltpu.sync_copy(ubuf0, acc.at[pl.ds(s * RPT + k * CS, CS), :])
            if phase == 0:
                pltpu.sync_copy(zbuf,
                                dacc.at[pl.ds(s * RPT + k * CS, CS), :])
        plsc.subcore_barrier()

        def _group(g, _):
            chunk0 = s * CH + g * K
            base = chunk0 * CS
            pltpu.sync_copy(srcq_hbm.at[pl.ds(q * MP + base, K * CS)], sidx)
            pltpu.sync_copy(dst3_hbm.at[pl.ds(chunk0, K), :, :], didx)

            def _issue(j):
                slot = j & 1
                gh = pltpu.async_copy(u_hbm.at[sidx.at[pl.ds(j * CS, CS)]],
                                      ubufs[slot], sems[slot])
                erow = q * (MP // 2) + (base + j * CS) // 2
                eh = pltpu.async_copy(e_hbm.at[pl.ds(erow, CS // 2), :],
                                      ebufs[slot], sems[slot])
                return gh, eh

            hs = [None, None]
            hs[0] = _issue(0)
            for j in range(K):
                slot = j & 1
                if j + 1 < K:
                    hs[(j + 1) & 1] = _issue(j + 1)
                gh, eh = hs[slot]
                gh.wait()
                eh.wait()
                ub = ubufs[slot]
                eb = ebufs[slot]

                @plsc.parallel_loop(0, CS // 2, unroll=2)
                def _row(i2, _ub=ub, _eb=eb):
                    for hp in range(2):
                        for jj in range(HQ // 16):
                            x = (_ub[i2 + hp * (CS // 2), pl.ds(jj * 16, 16)]
                                 + _eb[i2, pl.ds(hp * HQ + jj * 16, 16)])
                            _ub[i2 + hp * (CS // 2), pl.ds(jj * 16, 16)] = (
                                jnp.maximum(x, 0.0))

                pltpu.sync_copy(ub, acc.at[didx.at[j, 0]], add=True)
                if phase == 0:
                    pltpu.sync_copy(obuf, dacc.at[didx.at[j, 0]], add=True)
            return 0
        lax.fori_loop(0, CH // K, _group, 0)
        plsc.subcore_barrier()

        pltpu.sync_copy(acc.at[pl.ds(s * RPT, RPT), :],
                        r_hbm.at[q, pl.ds(s * RPT, RPT), :])
        if phase == 0:
            pltpu.sync_copy(dacc.at[pl.ds(s * RPT, RPT), :],
                            d_hbm.at[c, pl.ds(s * RPT, RPT), :])


def kernel(h, edge_index, edge_attr, W1, b1, W2, b2, W3, b3, W4, b4,
           gamma, beta):
    src = edge_index[0].astype(jnp.int32)
    dst = edge_index[1].astype(jnp.int32)
    pad = MP - M
    srcp = jnp.concatenate([src, jnp.full((pad,), N, jnp.int32)])
    dstp = jnp.concatenate([dst, jnp.full((pad,), N, jnp.int32)])
    h_pad = jnp.concatenate([h, jnp.zeros((NP - N, H), jnp.float32)], axis=0)
    ea_pad = jnp.concatenate(
        [edge_attr, jnp.zeros((pad, edge_attr.shape[1]), jnp.float32)], axis=0)

    u = pl.pallas_call(
        _u_body,
        grid=(NP // 128,),
        in_specs=[pl.BlockSpec((128, H), lambda i: (i, 0)),
                  pl.BlockSpec((H, H), lambda i: (0, 0)),
                  pl.BlockSpec((1, H), lambda i: (0, 0))],
        out_specs=pl.BlockSpec((4, 128, HQ), lambda i: (0, i, 0)),
        out_shape=jax.ShapeDtypeStruct((4, NP, HQ), jnp.float32),
    )(h_pad, W1[:H], b1.reshape(1, H))
    u_flat = u.reshape(4 * NP, HQ)

    w1b_q = W1[H:].reshape(16, 4, HQ).transpose(1, 0, 2)
    w1b_d = jnp.zeros((4, 32, 128), jnp.float32)
    w1b_d = w1b_d.at[:, :16, :HQ].set(w1b_q)
    w1b_d = w1b_d.at[:, 16:, HQ:].set(w1b_q)
    eab = ea_pad.reshape(MP // CS, CS, 16)
    a2 = jnp.concatenate([eab[:, :CS // 2, :], eab[:, CS // 2:, :]],
                         axis=2).reshape(MP // 2, 32)
    EB = 512
    MH = MP // 2
    e_pair = pl.pallas_call(
        _e_body,
        grid=(MH // EB, 4),
        in_specs=[pl.BlockSpec((EB, 32), lambda i, q: (i, 0)),
                  pl.BlockSpec((1, 32, 128), lambda i, q: (q, 0, 0))],
        out_specs=pl.BlockSpec((EB, 128),
                               lambda i, q: (q * (MH // EB) + i, 0)),
        out_shape=jax.ShapeDtypeStruct((2 * MP, 128), jnp.float32),
    )(a2, w1b_d)

    srcq = jnp.concatenate([srcp + qq * NP for qq in range(4)])
    dst3 = dstp.reshape(MP // CS, 1, CS)

    mesh = plsc.VectorSubcoreMesh(core_axis_name="c", subcore_axis_name="s")
    r, d = pl.kernel(
        _sc_body,
        mesh=mesh,
        compiler_params=pltpu.CompilerParams(use_tc_tiling_on_sc=False),
        out_type=[jax.ShapeDtypeStruct((4, NP, HQ), jnp.float32),
                  jax.ShapeDtypeStruct((2, NP, 16), jnp.float32)],
        scratch_types=[
            pltpu.VMEM((K * CS,), jnp.int32),
            pltpu.VMEM((K, 1, CS), jnp.int32),
            pltpu.VMEM((CS, HQ), jnp.float32),
            pltpu.VMEM((CS, HQ), jnp.float32),
            pltpu.VMEM((CS // 2, 2 * HQ), jnp.float32),
            pltpu.VMEM((CS // 2, 2 * HQ), jnp.float32),
            pltpu.VMEM((CS, 16), jnp.float32),
            pltpu.VMEM((CS, 16), jnp.float32),
            pltpu.VMEM_SHARED((NP, HQ), jnp.float32),
            pltpu.VMEM_SHARED((NP, 16), jnp.float32),
            pltpu.SemaphoreType.DMA,
            pltpu.SemaphoreType.DMA,
        ],
    )(u_flat, e_pair, srcq, dst3)

    out = pl.pallas_call(
        _ep_body,
        grid=(NP // 128,),
        in_specs=[
            pl.BlockSpec((4, 128, HQ), lambda i: (0, i, 0)),
            pl.BlockSpec((2, 128, 16), lambda i: (0, i, 0)),
            pl.BlockSpec((128, H), lambda i: (i, 0)),
            pl.BlockSpec((H, H), lambda i: (0, 0)),
            pl.BlockSpec((1, H), lambda i: (0, 0)),
            pl.BlockSpec((H, H), lambda i: (0, 0)),
            pl.BlockSpec((H, H), lambda i: (0, 0)),
            pl.BlockSpec((1, H), lambda i: (0, 0)),
            pl.BlockSpec((H, H), lambda i: (0, 0)),
            pl.BlockSpec((1, H), lambda i: (0, 0)),
            pl.BlockSpec((1, H), lambda i: (0, 0)),
            pl.BlockSpec((1, H), lambda i: (0, 0)),
        ],
        out_specs=pl.BlockSpec((128, H), lambda i: (i, 0)),
        out_shape=jax.ShapeDtypeStruct((NP, H), jnp.float32),
    )(r, d, h_pad, W2, b2.reshape(1, H), W3[:H], W3[H:], b3.reshape(1, H),
      W4, b4.reshape(1, H), gamma.reshape(1, H), beta.reshape(1, H))
    return out[:N]


# speedup vs baseline: 2.3132x; 1.1717x over previous
import functools

import jax
import jax.numpy as jnp
from jax import lax
from jax.experimental import pallas as pl
from jax.experimental.pallas import tpu as pltpu
from jax.experimental.pallas import tpu_sc as plsc

N = 10000
NP = 10240
H = 256
HQ = 64
M = 160000
CS = 128
CH = 80
MP = 16 * CH * CS
K = 8
RPT = NP // 16


def _u_body(h_ref, w_ref, b_ref, o_ref):
    p = jnp.dot(h_ref[...], w_ref[...], preferred_element_type=jnp.float32, precision=lax.Precision.HIGHEST)
    p = p + b_ref[...]
    o_ref[...] = jnp.stack([p[:, 0:64], p[:, 64:128],
                            p[:, 128:192], p[:, 192:256]], axis=0)


def _e_body(a_ref, w_ref, o_ref):
    o_ref[...] = jnp.dot(a_ref[...].astype(jnp.bfloat16),
                         w_ref[0].astype(jnp.bfloat16),
                         preferred_element_type=jnp.float32)


def _ep_body(r_ref, d_ref, h_ref, w2_ref, b2_ref, w3a_ref, w3b_ref, b3_ref,
             w4_ref, b4_ref, g_ref, be_ref, o_ref):
    rb = jnp.concatenate([r_ref[0], r_ref[1], r_ref[2], r_ref[3]], axis=1)
    deg = d_ref[0][:, 0:1]
    agg = jnp.dot(rb, w2_ref[...], preferred_element_type=jnp.float32, precision=lax.Precision.HIGHEST)
    agg = agg + deg * b2_ref[...]
    z = jnp.dot(h_ref[...], w3a_ref[...], preferred_element_type=jnp.float32, precision=lax.Precision.HIGHEST)
    z = z + jnp.dot(agg, w3b_ref[...], preferred_element_type=jnp.float32, precision=lax.Precision.HIGHEST)
    z = z + b3_ref[...]
    a1 = jnp.maximum(z, 0.0)
    dh = jnp.dot(a1, w4_ref[...], preferred_element_type=jnp.float32, precision=lax.Precision.HIGHEST)
    dh = dh + b4_ref[...]
    y = h_ref[...] + dh
    mu = jnp.mean(y, axis=1, keepdims=True)
    d0 = y - mu
    var = jnp.mean(d0 * d0, axis=1, keepdims=True)
    o_ref[...] = d0 * lax.rsqrt(var + 1e-5) * g_ref[...] + be_ref[...]


def _sc_body(u_hbm, e_hbm, srcq_hbm, dst3_hbm, r_hbm, d_hbm,
             sidx, didx, ubuf0, ubuf1, ebuf0, ebuf1, obuf, zbuf,
             acc, dacc, sem0, sem1):
    c = lax.axis_index("c")
    s = lax.axis_index("s")

    @plsc.parallel_loop(0, CS, unroll=4)
    def _init_row(i):
        obuf[i, :] = jnp.ones((16,), jnp.float32)
        zbuf[i, :] = jnp.zeros((16,), jnp.float32)

    ubufs = (ubuf0, ubuf1)
    ebufs = (ebuf0, ebuf1)
    sems = (sem0, sem1)

    for phase in range(2):
        q = c * 2 + phase

        @plsc.parallel_loop(0, CS, unroll=4)
        def _zero_row(i):
            for j in range(HQ // 16):
                ubuf0[i, pl.ds(j * 16, 16)] = jnp.zeros((16,), jnp.float32)
        for k in range(RPT // CS):
            pltpu.sync_copy(ubuf0, acc.at[pl.ds(s * RPT + k * CS, CS), :])
            if phase == 0:
                pltpu.sync_copy(zbuf,
                                dacc.at[pl.ds(s * RPT + k * CS, CS), :])
        plsc.subcore_barrier()

        def _group(g, _):
            chunk0 = s * CH + g * K
            base = chunk0 * CS
            pltpu.sync_copy(srcq_hbm.at[pl.ds(q * MP + base, K * CS)], sidx)
            pltpu.sync_copy(dst3_hbm.at[pl.ds(chunk0, K), :, :], didx)

            def _issue(j):
                slot = j & 1
                gh = pltpu.async_copy(u_hbm.at[sidx.at[pl.ds(j * CS, CS)]],
                                      ubufs[slot], sems[slot])
                erow = q * (MP // 2) + (base + j * CS) // 2
                eh = pltpu.async_copy(e_hbm.at[pl.ds(erow, CS // 2), :],
                                      ebufs[slot], sems[slot])
                return gh, eh

            hs = [None, None]
            hs[0] = _issue(0)
            for j in range(K):
                slot = j & 1
                if j + 1 < K:
                    hs[(j + 1) & 1] = _issue(j + 1)
                gh, eh = hs[slot]
                gh.wait()
                eh.wait()
                ub = ubufs[slot]
                eb = ebufs[slot]

                @plsc.parallel_loop(0, CS // 2, unroll=2)
                def _row(i2, _ub=ub, _eb=eb):
                    for hp in range(2):
                        for jj in range(HQ // 16):
                            x = (_ub[i2 + hp * (CS // 2), pl.ds(jj * 16, 16)]
                                 + _eb[i2, pl.ds(hp * HQ + jj * 16, 16)])
                            _ub[i2 + hp * (CS // 2), pl.ds(jj * 16, 16)] = (
                                jnp.maximum(x, 0.0))

                pltpu.sync_copy(ub, acc.at[didx.at[j, 0]], add=True)
                if phase == 0:
                    pltpu.sync_copy(obuf, dacc.at[didx.at[j, 0]], add=True)
            return 0
        lax.fori_loop(0, CH // K, _group, 0)
        plsc.subcore_barrier()

        pltpu.sync_copy(acc.at[pl.ds(s * RPT, RPT), :],
                        r_hbm.at[q, pl.ds(s * RPT, RPT), :])
        if phase == 0:
            pltpu.sync_copy(dacc.at[pl.ds(s * RPT, RPT), :],
                            d_hbm.at[c, pl.ds(s * RPT, RPT), :])


def kernel(h, edge_index, edge_attr, W1, b1, W2, b2, W3, b3, W4, b4,
           gamma, beta):
    src = edge_index[0].astype(jnp.int32)
    dst = edge_index[1].astype(jnp.int32)
    pad = MP - M
    srcp = jnp.concatenate([src, jnp.full((pad,), N, jnp.int32)])
    dstp = jnp.concatenate([dst, jnp.full((pad,), N, jnp.int32)])
    h_pad = jnp.concatenate([h, jnp.zeros((NP - N, H), jnp.float32)], axis=0)
    ea_pad = jnp.concatenate(
        [edge_attr, jnp.zeros((pad, edge_attr.shape[1]), jnp.float32)], axis=0)

    u = pl.pallas_call(
        _u_body,
        grid=(NP // 128,),
        in_specs=[pl.BlockSpec((128, H), lambda i: (i, 0)),
                  pl.BlockSpec((H, H), lambda i: (0, 0)),
                  pl.BlockSpec((1, H), lambda i: (0, 0))],
        out_specs=pl.BlockSpec((4, 128, HQ), lambda i: (0, i, 0)),
        out_shape=jax.ShapeDtypeStruct((4, NP, HQ), jnp.float32),
    )(h_pad, W1[:H], b1.reshape(1, H))
    u_flat = u.reshape(4 * NP, HQ)

    w1b_q = W1[H:].reshape(16, 4, HQ).transpose(1, 0, 2)
    w1b_d = jnp.zeros((4, 32, 128), jnp.float32)
    w1b_d = w1b_d.at[:, :16, :HQ].set(w1b_q)
    w1b_d = w1b_d.at[:, 16:, HQ:].set(w1b_q)
    eab = ea_pad.reshape(MP // CS, CS, 16)
    a2 = jnp.concatenate([eab[:, :CS // 2, :], eab[:, CS // 2:, :]],
                         axis=2).reshape(MP // 2, 32)
    EB = 1024
    MH = MP // 2
    e_pair = pl.pallas_call(
        _e_body,
        grid=(MH // EB, 4),
        in_specs=[pl.BlockSpec((EB, 32), lambda i, q: (i, 0)),
                  pl.BlockSpec((1, 32, 128), lambda i, q: (q, 0, 0))],
        out_specs=pl.BlockSpec((EB, 128),
                               lambda i, q: (q * (MH // EB) + i, 0)),
        out_shape=jax.ShapeDtypeStruct((2 * MP, 128), jnp.float32),
    )(a2, w1b_d)

    srcq = jnp.concatenate([srcp + qq * NP for qq in range(4)])
    dst3 = dstp.reshape(MP // CS, 1, CS)

    mesh = plsc.VectorSubcoreMesh(core_axis_name="c", subcore_axis_name="s")
    r, d = pl.kernel(
        _sc_body,
        mesh=mesh,
        compiler_params=pltpu.CompilerParams(use_tc_tiling_on_sc=False),
        out_type=[jax.ShapeDtypeStruct((4, NP, HQ), jnp.float32),
                  jax.ShapeDtypeStruct((2, NP, 16), jnp.float32)],
        scratch_types=[
            pltpu.VMEM((K * CS,), jnp.int32),
            pltpu.VMEM((K, 1, CS), jnp.int32),
            pltpu.VMEM((CS, HQ), jnp.float32),
            pltpu.VMEM((CS, HQ), jnp.float32),
            pltpu.VMEM((CS // 2, 2 * HQ), jnp.float32),
            pltpu.VMEM((CS // 2, 2 * HQ), jnp.float32),
            pltpu.VMEM((CS, 16), jnp.float32),
            pltpu.VMEM((CS, 16), jnp.float32),
            pltpu.VMEM_SHARED((NP, HQ), jnp.float32),
            pltpu.VMEM_SHARED((NP, 16), jnp.float32),
            pltpu.SemaphoreType.DMA,
            pltpu.SemaphoreType.DMA,
        ],
    )(u_flat, e_pair, srcq, dst3)

    out = pl.pallas_call(
        _ep_body,
        grid=(NP // 128,),
        in_specs=[
            pl.BlockSpec((4, 128, HQ), lambda i: (0, i, 0)),
            pl.BlockSpec((2, 128, 16), lambda i: (0, i, 0)),
            pl.BlockSpec((128, H), lambda i: (i, 0)),
            pl.BlockSpec((H, H), lambda i: (0, 0)),
            pl.BlockSpec((1, H), lambda i: (0, 0)),
            pl.BlockSpec((H, H), lambda i: (0, 0)),
            pl.BlockSpec((H, H), lambda i: (0, 0)),
            pl.BlockSpec((1, H), lambda i: (0, 0)),
            pl.BlockSpec((H, H), lambda i: (0, 0)),
            pl.BlockSpec((1, H), lambda i: (0, 0)),
            pl.BlockSpec((1, H), lambda i: (0, 0)),
            pl.BlockSpec((1, H), lambda i: (0, 0)),
        ],
        out_specs=pl.BlockSpec((128, H), lambda i: (i, 0)),
        out_shape=jax.ShapeDtypeStruct((NP, H), jnp.float32),
    )(r, d, h_pad, W2, b2.reshape(1, H), W3[:H], W3[H:], b3.reshape(1, H),
      W4, b4.reshape(1, H), gamma.reshape(1, H), beta.reshape(1, H))
    return out[:N]


# speedup vs baseline: 2.3212x; 1.0035x over previous
import functools

import jax
import jax.numpy as jnp
from jax import lax
from jax.experimental import pallas as pl
from jax.experimental.pallas import tpu as pltpu
from jax.experimental.pallas import tpu_sc as plsc

N = 10000
NP = 10240
H = 256
HQ = 64
M = 160000
CS = 128
CH = 80
MP = 16 * CH * CS
K = 8
RPT = NP // 16


def _u_body(h_ref, w_ref, b_ref, o_ref):
    p = jnp.dot(h_ref[...], w_ref[...], preferred_element_type=jnp.float32, precision=lax.Precision.HIGHEST)
    p = p + b_ref[...]
    o_ref[...] = jnp.stack([p[:, 0:64], p[:, 64:128],
                            p[:, 128:192], p[:, 192:256]], axis=0)


def _e_body(a_ref, w_ref, o_ref):
    o_ref[...] = jnp.dot(a_ref[...].astype(jnp.bfloat16),
                         w_ref[0].astype(jnp.bfloat16),
                         preferred_element_type=jnp.float32)


def _ep_body(r_ref, d_ref, h_ref, w2_ref, b2_ref, w3a_ref, w3b_ref, b3_ref,
             w4_ref, b4_ref, g_ref, be_ref, o_ref):
    rb = jnp.concatenate([r_ref[0], r_ref[1], r_ref[2], r_ref[3]], axis=1)
    deg = d_ref[0][:, 0:1]
    agg = jnp.dot(rb, w2_ref[...], preferred_element_type=jnp.float32, precision=lax.Precision.HIGHEST)
    agg = agg + deg * b2_ref[...]
    z = jnp.dot(h_ref[...], w3a_ref[...], preferred_element_type=jnp.float32, precision=lax.Precision.HIGHEST)
    z = z + jnp.dot(agg, w3b_ref[...], preferred_element_type=jnp.float32, precision=lax.Precision.HIGHEST)
    z = z + b3_ref[...]
    a1 = jnp.maximum(z, 0.0)
    dh = jnp.dot(a1, w4_ref[...], preferred_element_type=jnp.float32, precision=lax.Precision.HIGHEST)
    dh = dh + b4_ref[...]
    y = h_ref[...] + dh
    mu = jnp.mean(y, axis=1, keepdims=True)
    d0 = y - mu
    var = jnp.mean(d0 * d0, axis=1, keepdims=True)
    o_ref[...] = d0 * lax.rsqrt(var + 1e-5) * g_ref[...] + be_ref[...]


def _sc_body(u_hbm, e_hbm, srcq_hbm, dst3_hbm, r_hbm, d_hbm,
             sidx, didx, ubuf0, ubuf1, ebuf0, ebuf1, obuf, zbuf,
             acc, dacc, sem0, sem1, ssem0, ssem1):
    c = lax.axis_index("c")
    s = lax.axis_index("s")

    @plsc.parallel_loop(0, CS, unroll=4)
    def _init_row(i):
        obuf[i, :] = jnp.ones((16,), jnp.float32)
        zbuf[i, :] = jnp.zeros((16,), jnp.float32)

    ubufs = (ubuf0, ubuf1)
    ebufs = (ebuf0, ebuf1)
    sems = (sem0, sem1)
    ssems = (ssem0, ssem1)

    for phase in range(2):
        q = c * 2 + phase

        @plsc.parallel_loop(0, CS, unroll=4)
        def _zero_row(i):
            for j in range(HQ // 16):
                ubuf0[i, pl.ds(j * 16, 16)] = jnp.zeros((16,), jnp.float32)
        for k in range(RPT // CS):
            pltpu.sync_copy(ubuf0, acc.at[pl.ds(s * RPT + k * CS, CS), :])
            if phase == 0:
                pltpu.sync_copy(zbuf,
                                dacc.at[pl.ds(s * RPT + k * CS, CS), :])
        plsc.subcore_barrier()

        def _group(g, _):
            chunk0 = s * CH + g * K
            base = chunk0 * CS
            pltpu.sync_copy(srcq_hbm.at[pl.ds(q * MP + base, K * CS)], sidx)
            pltpu.sync_copy(dst3_hbm.at[pl.ds(chunk0, K), :, :], didx)

            def _issue(j):
                slot = j & 1
                gh = pltpu.async_copy(u_hbm.at[sidx.at[pl.ds(j * CS, CS)]],
                                      ubufs[slot], sems[slot])
                erow = q * (MP // 2) + (base + j * CS) // 2
                eh = pltpu.async_copy(e_hbm.at[pl.ds(erow, CS // 2), :],
                                      ebufs[slot], sems[slot])
                return gh, eh

            hs = [None, None]
            sc_hs = [None, None]
            hs[0] = _issue(0)
            for j in range(K):
                slot = j & 1
                if j + 1 < K:
                    nslot = (j + 1) & 1
                    if sc_hs[nslot] is not None:
                        for ww in sc_hs[nslot]:
                            ww.wait()
                        sc_hs[nslot] = None
                    hs[nslot] = _issue(j + 1)
                gh, eh = hs[slot]
                gh.wait()
                eh.wait()
                ub = ubufs[slot]
                eb = ebufs[slot]

                @plsc.parallel_loop(0, CS // 2, unroll=2)
                def _row(i2, _ub=ub, _eb=eb):
                    for hp in range(2):
                        for jj in range(HQ // 16):
                            x = (_ub[i2 + hp * (CS // 2), pl.ds(jj * 16, 16)]
                                 + _eb[i2, pl.ds(hp * HQ + jj * 16, 16)])
                            _ub[i2 + hp * (CS // 2), pl.ds(jj * 16, 16)] = (
                                jnp.maximum(x, 0.0))

                ws = [pltpu.async_copy(ub, acc.at[didx.at[j, 0]],
                                       ssems[slot], add=True)]
                if phase == 0:
                    ws.append(pltpu.async_copy(obuf, dacc.at[didx.at[j, 0]],
                                               ssems[slot], add=True))
                sc_hs[slot] = ws
            for pend in sc_hs:
                if pend is not None:
                    for ww in pend:
                        ww.wait()
            return 0
        lax.fori_loop(0, CH // K, _group, 0)
        plsc.subcore_barrier()

        pltpu.sync_copy(acc.at[pl.ds(s * RPT, RPT), :],
                        r_hbm.at[q, pl.ds(s * RPT, RPT), :])
        if phase == 0:
            pltpu.sync_copy(dacc.at[pl.ds(s * RPT, RPT), :],
                            d_hbm.at[c, pl.ds(s * RPT, RPT), :])


def kernel(h, edge_index, edge_attr, W1, b1, W2, b2, W3, b3, W4, b4,
           gamma, beta):
    src = edge_index[0].astype(jnp.int32)
    dst = edge_index[1].astype(jnp.int32)
    pad = MP - M
    srcp = jnp.concatenate([src, jnp.full((pad,), N, jnp.int32)])
    dstp = jnp.concatenate([dst, jnp.full((pad,), N, jnp.int32)])
    h_pad = jnp.concatenate([h, jnp.zeros((NP - N, H), jnp.float32)], axis=0)
    ea_pad = jnp.concatenate(
        [edge_attr, jnp.zeros((pad, edge_attr.shape[1]), jnp.float32)], axis=0)

    u = pl.pallas_call(
        _u_body,
        grid=(NP // 128,),
        in_specs=[pl.BlockSpec((128, H), lambda i: (i, 0)),
                  pl.BlockSpec((H, H), lambda i: (0, 0)),
                  pl.BlockSpec((1, H), lambda i: (0, 0))],
        out_specs=pl.BlockSpec((4, 128, HQ), lambda i: (0, i, 0)),
        out_shape=jax.ShapeDtypeStruct((4, NP, HQ), jnp.float32),
    )(h_pad, W1[:H], b1.reshape(1, H))
    u_flat = u.reshape(4 * NP, HQ)

    w1b_q = W1[H:].reshape(16, 4, HQ).transpose(1, 0, 2)
    w1b_d = jnp.zeros((4, 32, 128), jnp.float32)
    w1b_d = w1b_d.at[:, :16, :HQ].set(w1b_q)
    w1b_d = w1b_d.at[:, 16:, HQ:].set(w1b_q)
    eab = ea_pad.reshape(MP // CS, CS, 16)
    a2 = jnp.concatenate([eab[:, :CS // 2, :], eab[:, CS // 2:, :]],
                         axis=2).reshape(MP // 2, 32)
    EB = 1024
    MH = MP // 2
    e_pair = pl.pallas_call(
        _e_body,
        grid=(MH // EB, 4),
        in_specs=[pl.BlockSpec((EB, 32), lambda i, q: (i, 0)),
                  pl.BlockSpec((1, 32, 128), lambda i, q: (q, 0, 0))],
        out_specs=pl.BlockSpec((EB, 128),
                               lambda i, q: (q * (MH // EB) + i, 0)),
        out_shape=jax.ShapeDtypeStruct((2 * MP, 128), jnp.float32),
    )(a2, w1b_d)

    srcq = jnp.concatenate([srcp + qq * NP for qq in range(4)])
    dst3 = dstp.reshape(MP // CS, 1, CS)

    mesh = plsc.VectorSubcoreMesh(core_axis_name="c", subcore_axis_name="s")
    r, d = pl.kernel(
        _sc_body,
        mesh=mesh,
        compiler_params=pltpu.CompilerParams(use_tc_tiling_on_sc=False),
        out_type=[jax.ShapeDtypeStruct((4, NP, HQ), jnp.float32),
                  jax.ShapeDtypeStruct((2, NP, 16), jnp.float32)],
        scratch_types=[
            pltpu.VMEM((K * CS,), jnp.int32),
            pltpu.VMEM((K, 1, CS), jnp.int32),
            pltpu.VMEM((CS, HQ), jnp.float32),
            pltpu.VMEM((CS, HQ), jnp.float32),
            pltpu.VMEM((CS // 2, 2 * HQ), jnp.float32),
            pltpu.VMEM((CS // 2, 2 * HQ), jnp.float32),
            pltpu.VMEM((CS, 16), jnp.float32),
            pltpu.VMEM((CS, 16), jnp.float32),
            pltpu.VMEM_SHARED((NP, HQ), jnp.float32),
            pltpu.VMEM_SHARED((NP, 16), jnp.float32),
            pltpu.SemaphoreType.DMA,
            pltpu.SemaphoreType.DMA,
            pltpu.SemaphoreType.DMA,
            pltpu.SemaphoreType.DMA,
        ],
    )(u_flat, e_pair, srcq, dst3)

    out = pl.pallas_call(
        _ep_body,
        grid=(NP // 128,),
        in_specs=[
            pl.BlockSpec((4, 128, HQ), lambda i: (0, i, 0)),
            pl.BlockSpec((2, 128, 16), lambda i: (0, i, 0)),
            pl.BlockSpec((128, H), lambda i: (i, 0)),
            pl.BlockSpec((H, H), lambda i: (0, 0)),
            pl.BlockSpec((1, H), lambda i: (0, 0)),
            pl.BlockSpec((H, H), lambda i: (0, 0)),
            pl.BlockSpec((H, H), lambda i: (0, 0)),
            pl.BlockSpec((1, H), lambda i: (0, 0)),
            pl.BlockSpec((H, H), lambda i: (0, 0)),
            pl.BlockSpec((1, H), lambda i: (0, 0)),
            pl.BlockSpec((1, H), lambda i: (0, 0)),
            pl.BlockSpec((1, H), lambda i: (0, 0)),
        ],
        out_specs=pl.BlockSpec((128, H), lambda i: (i, 0)),
        out_shape=jax.ShapeDtypeStruct((NP, H), jnp.float32),
    )(r, d, h_pad, W2, b2.reshape(1, H), W3[:H], W3[H:], b3.reshape(1, H),
      W4, b4.reshape(1, H), gamma.reshape(1, H), beta.reshape(1, H))
    return out[:N]
